# baseline (speedup 1.0000x reference)
import functools

import jax
import jax.numpy as jnp
from jax import lax
from jax.experimental import pallas as pl
from jax.experimental.pallas import tpu as pltpu
from jax.experimental.pallas import tpu_sc as plsc

_B = 4096
_L = 200
_D = 64
_LANES = 16
_NC = 2
_NS = 16
_NW = _NC * _NS
_SEQ_PER_W = _B // _NW
_G = 2
_CH = _G * _L
_NCHUNK = _SEQ_PER_W // _G


def _emb_body(x_hbm, tab_hbm, pos_hbm, out_hbm, pos_v, idx_v, rows_v, sem):
    cid = lax.axis_index("c")
    sid = lax.axis_index("s")
    wid = sid * _NC + cid
    base = wid * (_SEQ_PER_W * _L)

    pltpu.sync_copy(pos_hbm, pos_v)

    @pl.loop(0, _NCHUNK)
    def _(ch):
        off = base + ch * _CH
        pltpu.sync_copy(x_hbm.at[pl.ds(off, _CH)], idx_v)
        pltpu.async_copy(tab_hbm.at[idx_v], rows_v, sem).wait()

        @pl.loop(0, _L)
        def _(r):
            for c in range(_D // _LANES):
                pv = pos_v[r, pl.ds(c * _LANES, _LANES)]
                for s in range(_G):
                    plsc.addupdate(
                        rows_v.at[s * _L + r, pl.ds(c * _LANES, _LANES)], pv)

        pltpu.sync_copy(rows_v, out_hbm.at[pl.ds(off, _CH)])


@jax.jit
def _emb(x_flat, table, pos):
    mesh = plsc.VectorSubcoreMesh(core_axis_name="c", subcore_axis_name="s")
    run = pl.kernel(
        _emb_body,
        out_type=jax.ShapeDtypeStruct((_B * _L, _D), jnp.float32),
        mesh=mesh,
        scratch_types=[
            pltpu.VMEM((_L, _D), jnp.float32),
            pltpu.VMEM((_CH,), jnp.int32),
            pltpu.VMEM((_CH, _D), jnp.float32),
            pltpu.SemaphoreType.DMA,
        ],
        compiler_params=pltpu.CompilerParams(use_tc_tiling_on_sc=False),
    )
    return run(x_flat, table, pos)


def kernel(x, table, pos_embed):
    x_flat = x.reshape(-1)
    pos = pos_embed.reshape(_L, _D)
    out = _emb(x_flat, table, pos)
    return out.reshape(_B, _L, _D)


# speedup vs baseline: 1.1006x; 1.1006x over previous
import functools

import jax
import jax.numpy as jnp
from jax import lax
from jax.experimental import pallas as pl
from jax.experimental.pallas import tpu as pltpu
from jax.experimental.pallas import tpu_sc as plsc

_B = 4096
_L = 200
_D = 64
_DP = 128
_LANES = 16
_NC = 2
_NS = 16
_NW = _NC * _NS
_SEQ_PER_W = _B // _NW
_G = 1
_CH = _G * _L
_NCHUNK = _SEQ_PER_W // _G


def _emb_body(x_hbm, tab_hbm, pos_hbm, out_hbm, pos_v, idx_v, rows_v, st_v,
              sem):
    cid = lax.axis_index("c")
    sid = lax.axis_index("s")
    wid = sid * _NC + cid
    base = wid * (_SEQ_PER_W * _L)

    pltpu.sync_copy(pos_hbm, pos_v)

    @pl.loop(0, _NCHUNK)
    def _(ch):
        off = base + ch * _CH
        pltpu.sync_copy(x_hbm.at[pl.ds(off, _CH)], idx_v)
        pltpu.async_copy(tab_hbm.at[idx_v], rows_v, sem).wait()

        @pl.loop(0, _L)
        def _(r):
            for c in range(_D // _LANES):
                sl = pl.ds(c * _LANES, _LANES)
                for s in range(_G):
                    st_v[s * _L + r, sl] = rows_v[s * _L + r, sl] + pos_v[r, sl]

        pltpu.sync_copy(st_v, out_hbm.at[pl.ds(off, _CH)])


@jax.jit
def _emb(x_flat, table_padded, pos):
    mesh = plsc.VectorSubcoreMesh(core_axis_name="c", subcore_axis_name="s")
    run = pl.kernel(
        _emb_body,
        out_type=jax.ShapeDtypeStruct((_B * _L, _D), jnp.float32),
        mesh=mesh,
        scratch_types=[
            pltpu.VMEM((_L, _D), jnp.float32),
            pltpu.VMEM((_CH,), jnp.int32),
            pltpu.VMEM((_CH, _DP), jnp.float32),
            pltpu.VMEM((_CH, _D), jnp.float32),
            pltpu.SemaphoreType.DMA,
        ],
    )
    return run(x_flat, table_padded, pos)


def kernel(x, table, pos_embed):
    x_flat = x.reshape(-1)
    table_padded = jnp.pad(table, ((0, 0), (0, _DP - _D)))
    pos = pos_embed.reshape(_L, _D)
    out = _emb(x_flat, table_padded, pos)
    return out.reshape(_B, _L, _D)


# speedup vs baseline: 1.1110x; 1.0094x over previous
import functools

import jax
import jax.numpy as jnp
from jax import lax
from jax.experimental import pallas as pl
from jax.experimental.pallas import tpu as pltpu
from jax.experimental.pallas import tpu_sc as plsc

_B = 4096
_L = 200
_D = 64
_LANES = 16
_NC = 2
_NS = 16
_NW = _NC * _NS
_SEQ_PER_W = _B // _NW
_ROWS_PER_W = _SEQ_PER_W * _L
_G = 1
_CH = _G * _L
_NCHUNK = _SEQ_PER_W // _G
_NB = 4


def _emb_body(x_hbm, tab_hbm, pos_hbm, out_hbm, pos_v, idx_v,
              r0, r1, r2, r3, isem, g0, g1, g2, g3, o0, o1, o2, o3):
    rows = (r0, r1, r2, r3)
    gsem = (g0, g1, g2, g3)
    osem = (o0, o1, o2, o3)
    cid = lax.axis_index("c")
    sid = lax.axis_index("s")
    wid = sid * _NC + cid
    base = wid * _ROWS_PER_W

    pltpu.sync_copy(pos_hbm, pos_v)
    pltpu.async_copy(x_hbm.at[pl.ds(base, _ROWS_PER_W)], idx_v, isem).wait()

    def gather(ch, j):
        return pltpu.make_async_copy(
            tab_hbm.at[idx_v.at[pl.ds(ch * _CH, _CH)]], rows[j], gsem[j])

    def outcp(ch, j):
        return pltpu.make_async_copy(
            rows[j], out_hbm.at[pl.ds(base + ch * _CH, _CH)], osem[j])

    gather(0, 0).start()
    gather(1, 1).start()

    @pl.loop(0, _NCHUNK // _NB)
    def _(k):
        for j in range(_NB):
            ch = k * _NB + j
            gather(ch, j).wait()

            @pl.loop(0, _L, step=4)
            def _(r):
                for rr in range(4):
                    for c in range(_D // _LANES):
                        sl = pl.ds(c * _LANES, _LANES)
                        plsc.addupdate(rows[j].at[r + rr, sl],
                                       pos_v[r + rr, sl])

            outcp(ch, j).start()
            jn = (j + 2) % _NB

            @pl.when(ch >= 2)
            def _():
                outcp(ch - 2, jn).wait()

            @pl.when(ch < _NCHUNK - 2)
            def _():
                gather(ch + 2, jn).start()

    for ch in (_NCHUNK - 2, _NCHUNK - 1):
        outcp(ch, ch % _NB).wait()


@jax.jit
def _emb(x_flat, table, pos):
    mesh = plsc.VectorSubcoreMesh(core_axis_name="c", subcore_axis_name="s")
    run = pl.kernel(
        _emb_body,
        out_type=jax.ShapeDtypeStruct((_B * _L, _D), jnp.float32),
        mesh=mesh,
        scratch_types=[
            pltpu.VMEM((_L, _D), jnp.float32),
            pltpu.VMEM((_ROWS_PER_W,), jnp.int32),
        ] + [pltpu.VMEM((_CH, _D), jnp.float32) for _ in range(_NB)]
          + [pltpu.SemaphoreType.DMA for _ in range(2 * _NB + 1)],
        compiler_params=pltpu.CompilerParams(use_tc_tiling_on_sc=False),
    )
    return run(x_flat, table, pos)


def kernel(x, table, pos_embed):
    x_flat = x.reshape(-1)
    pos = pos_embed.reshape(_L, _D)
    out = _emb(x_flat, table, pos)
    return out.reshape(_B, _L, _D)
